# baseline (speedup 1.0000x reference)
import functools

import jax
import jax.numpy as jnp
from jax import lax
from jax.experimental import pallas as pl
from jax.experimental.pallas import tpu as pltpu
from jax.experimental.pallas import tpu_sc as plsc

EMBED_DIM = 32
HIDDEN_DIM = 128
BATCH = 16384
N_CHUNKS = 2
CHUNK = BATCH // N_CHUNKS

NC = 2
NS = 16
NW = NC * NS
B_PER_W = CHUNK // NW

BLK = 2048


TILE_W = 128
HALF = 8


def _sc_gather(user, movie, utabT, mtabT):
    mesh = plsc.VectorSubcoreMesh(
        core_axis_name="c", subcore_axis_name="s", num_cores=NC, num_subcores=NS
    )
    n_pairs = B_PER_W // 16

    @functools.partial(
        pl.kernel,
        mesh=mesh,
        compiler_params=pltpu.CompilerParams(needs_layout_passes=False),
        out_type=[
            jax.ShapeDtypeStruct((EMBED_DIM, CHUNK), jnp.float32),
            jax.ShapeDtypeStruct((EMBED_DIM, CHUNK), jnp.float32),
        ],
        scratch_types=[
            pltpu.VMEM((B_PER_W,), jnp.int32),
            pltpu.VMEM((B_PER_W,), jnp.int32),
            pltpu.VMEM((16, EMBED_DIM, TILE_W), jnp.float32),
            pltpu.VMEM((EMBED_DIM, B_PER_W), jnp.float32),
            pltpu.SemaphoreType.DMA,
            pltpu.SemaphoreType.DMA,
        ],
    )
    def gather_kernel(user_hbm, movie_hbm, utabT_hbm, mtabT_hbm,
                      ueT_hbm, meT_hbm,
                      uidx_v, midx_v, bufs_v, cols_v, sem_a, sem_b):
        wid = lax.axis_index("s") * NC + lax.axis_index("c")
        base = wid * B_PER_W
        pltpu.sync_copy(user_hbm.at[pl.ds(base, B_PER_W)], uidx_v)
        pltpu.sync_copy(movie_hbm.at[pl.ds(base, B_PER_W)], midx_v)

        rows0 = lax.broadcasted_iota(jnp.int32, (16,), 0)
        rows1 = rows0 + 16
        sems = (sem_a, sem_b)

        for idx_v, tabT_hbm, outT_hbm in (
            (uidx_v, utabT_hbm, ueT_hbm),
            (midx_v, mtabT_hbm, meT_hbm),
        ):
            def fire(iv, half, tabT_hbm=tabT_hbm):
                for i in range(HALF):
                    s = iv[half * HALF + i]
                    q = pl.multiple_of((s >> 7) << 7, TILE_W)
                    pltpu.async_copy(
                        tabT_hbm.at[:, pl.ds(q, TILE_W)],
                        bufs_v.at[half * HALF + i], sems[half])

            def drain(half, tabT_hbm=tabT_hbm):
                for _ in range(HALF):
                    pltpu.make_async_copy(
                        tabT_hbm.at[:, pl.ds(0, TILE_W)],
                        bufs_v.at[0], sems[half]).wait()

            def extract(iv, half, jbase):
                for i in range(HALF):
                    s = iv[half * HALF + i]
                    lane = jnp.broadcast_to(s & 127, (16,))
                    jcol = jnp.broadcast_to(jbase + (half * HALF + i), (16,))
                    slot = bufs_v.at[half * HALF + i]
                    v0 = plsc.load_gather(slot, [rows0, lane])
                    v1 = plsc.load_gather(slot, [rows1, lane])
                    plsc.store_scatter(cols_v, [rows0, jcol], v0)
                    plsc.store_scatter(cols_v, [rows1, jcol], v1)

            iv0 = idx_v[pl.ds(0, 16)]
            fire(iv0, 0)
            fire(iv0, 1)

            def pair(p, carry, idx_v=idx_v):
                offp = pl.multiple_of(p * 16, 16)
                ivprev = idx_v[pl.ds(offp - 16, 16)]
                ivcur = idx_v[pl.ds(offp, 16)]
                drain(0)
                extract(ivprev, 0, offp - 16)
                fire(ivcur, 0)
                drain(1)
                extract(ivprev, 1, offp - 16)
                fire(ivcur, 1)
                return carry

            lax.fori_loop(1, n_pairs, pair, 0)

            ivlast = idx_v[pl.ds(B_PER_W - 16, 16)]
            drain(0)
            extract(ivlast, 0, B_PER_W - 16)
            drain(1)
            extract(ivlast, 1, B_PER_W - 16)
            pltpu.sync_copy(cols_v, outT_hbm.at[:, pl.ds(base, B_PER_W)])

    return gather_kernel(user, movie, utabT, mtabT)


def _mlp_body(ueT_ref, meT_ref, w1_ref, b1_ref, w2_ref, b2_ref, w3_ref, b3_ref,
              out_ref):
    w1 = w1_ref[...]
    hT = (
        lax.dot_general(w1[:EMBED_DIM], ueT_ref[...],
                        (((0,), (0,)), ((), ())),
                        preferred_element_type=jnp.float32)
        + lax.dot_general(w1[EMBED_DIM:], meT_ref[...],
                          (((0,), (0,)), ((), ())),
                          preferred_element_type=jnp.float32)
        + b1_ref[...]
    )
    hT = jnp.maximum(hT, 0.0)
    h2T = lax.dot_general(w2_ref[...], hT, (((0,), (0,)), ((), ())),
                          preferred_element_type=jnp.float32) + b2_ref[...]
    h2T = jnp.maximum(h2T, 0.0)
    s = jnp.sum(h2T * w3_ref[...], axis=0) + b3_ref[0]
    out_ref[...] = 1.0 / (1.0 + jnp.exp(-s))


def _tc_mlp(ueT, meT, W1, b1, W2, b2, W3, b3):
    b1c = b1.reshape(HIDDEN_DIM, 1)
    b2c = b2.reshape(HIDDEN_DIM // 2, 1)
    grid = (CHUNK // BLK,)
    full = lambda shape: pl.BlockSpec(shape, lambda i: tuple(0 for _ in shape))
    return pl.pallas_call(
        _mlp_body,
        grid=grid,
        in_specs=[
            pl.BlockSpec((EMBED_DIM, BLK), lambda i: (0, i)),
            pl.BlockSpec((EMBED_DIM, BLK), lambda i: (0, i)),
            full((2 * EMBED_DIM, HIDDEN_DIM)),
            full((HIDDEN_DIM, 1)),
            full((HIDDEN_DIM, HIDDEN_DIM // 2)),
            full((HIDDEN_DIM // 2, 1)),
            full((HIDDEN_DIM // 2, 1)),
            pl.BlockSpec(memory_space=pltpu.SMEM),
        ],
        out_specs=pl.BlockSpec((BLK,), lambda i: (i,)),
        out_shape=jax.ShapeDtypeStruct((CHUNK,), jnp.float32),
    )(ueT, meT, W1, b1c, W2, b2c, W3, b3)


def kernel(user, movie, user_table, movie_table, W1, b1, W2, b2, W3, b3):
    user = user.astype(jnp.int32)
    movie = movie.astype(jnp.int32)
    utabT = user_table.T
    mtabT = movie_table.T
    embs = [
        _sc_gather(user[c * CHUNK:(c + 1) * CHUNK],
                   movie[c * CHUNK:(c + 1) * CHUNK], utabT, mtabT)
        for c in range(N_CHUNKS)
    ]
    outs = [_tc_mlp(ueT, meT, W1, b1, W2, b2, W3, b3) for ueT, meT in embs]
    return jnp.concatenate(outs)


# speedup vs baseline: 1.0303x; 1.0303x over previous
import functools

import jax
import jax.numpy as jnp
from jax import lax
from jax.experimental import pallas as pl
from jax.experimental.pallas import tpu as pltpu
from jax.experimental.pallas import tpu_sc as plsc

EMBED_DIM = 32
HIDDEN_DIM = 128
BATCH = 16384
N_CHUNKS = 1
CHUNK = BATCH // N_CHUNKS

NC = 2
NS = 16
NW = NC * NS
B_PER_W = CHUNK // NW

BLK = 2048


TILE_W = 128
HALF = 8


def _sc_gather(user, movie, utabT, mtabT):
    mesh = plsc.VectorSubcoreMesh(
        core_axis_name="c", subcore_axis_name="s", num_cores=NC, num_subcores=NS
    )
    n_pairs = B_PER_W // 16

    @functools.partial(
        pl.kernel,
        mesh=mesh,
        compiler_params=pltpu.CompilerParams(needs_layout_passes=False),
        out_type=[
            jax.ShapeDtypeStruct((EMBED_DIM, CHUNK), jnp.float32),
            jax.ShapeDtypeStruct((EMBED_DIM, CHUNK), jnp.float32),
        ],
        scratch_types=[
            pltpu.VMEM((B_PER_W,), jnp.int32),
            pltpu.VMEM((B_PER_W,), jnp.int32),
            pltpu.VMEM((16, EMBED_DIM, TILE_W), jnp.float32),
            pltpu.VMEM((EMBED_DIM, B_PER_W), jnp.float32),
            pltpu.SemaphoreType.DMA,
            pltpu.SemaphoreType.DMA,
        ],
    )
    def gather_kernel(user_hbm, movie_hbm, utabT_hbm, mtabT_hbm,
                      ueT_hbm, meT_hbm,
                      uidx_v, midx_v, bufs_v, cols_v, sem_a, sem_b):
        wid = lax.axis_index("s") * NC + lax.axis_index("c")
        base = wid * B_PER_W
        pltpu.sync_copy(user_hbm.at[pl.ds(base, B_PER_W)], uidx_v)
        pltpu.sync_copy(movie_hbm.at[pl.ds(base, B_PER_W)], midx_v)

        rows0 = lax.broadcasted_iota(jnp.int32, (16,), 0)
        rows1 = rows0 + 16
        sems = (sem_a, sem_b)

        for idx_v, tabT_hbm, outT_hbm in (
            (uidx_v, utabT_hbm, ueT_hbm),
            (midx_v, mtabT_hbm, meT_hbm),
        ):
            def fire(iv, half, tabT_hbm=tabT_hbm):
                for i in range(HALF):
                    s = iv[half * HALF + i]
                    q = pl.multiple_of((s >> 7) << 7, TILE_W)
                    pltpu.async_copy(
                        tabT_hbm.at[:, pl.ds(q, TILE_W)],
                        bufs_v.at[half * HALF + i], sems[half])

            def drain(half, tabT_hbm=tabT_hbm):
                for _ in range(HALF):
                    pltpu.make_async_copy(
                        tabT_hbm.at[:, pl.ds(0, TILE_W)],
                        bufs_v.at[0], sems[half]).wait()

            def extract(iv, half, jbase):
                for i in range(HALF):
                    s = iv[half * HALF + i]
                    lane = jnp.broadcast_to(s & 127, (16,))
                    jcol = jnp.broadcast_to(jbase + (half * HALF + i), (16,))
                    slot = bufs_v.at[half * HALF + i]
                    v0 = plsc.load_gather(slot, [rows0, lane])
                    v1 = plsc.load_gather(slot, [rows1, lane])
                    plsc.store_scatter(cols_v, [rows0, jcol], v0)
                    plsc.store_scatter(cols_v, [rows1, jcol], v1)

            iv0 = idx_v[pl.ds(0, 16)]
            fire(iv0, 0)
            fire(iv0, 1)

            def pair(p, carry, idx_v=idx_v):
                offp = pl.multiple_of(p * 16, 16)
                ivprev = idx_v[pl.ds(offp - 16, 16)]
                ivcur = idx_v[pl.ds(offp, 16)]
                drain(0)
                extract(ivprev, 0, offp - 16)
                fire(ivcur, 0)
                drain(1)
                extract(ivprev, 1, offp - 16)
                fire(ivcur, 1)
                return carry

            lax.fori_loop(1, n_pairs, pair, 0)

            ivlast = idx_v[pl.ds(B_PER_W - 16, 16)]
            drain(0)
            extract(ivlast, 0, B_PER_W - 16)
            drain(1)
            extract(ivlast, 1, B_PER_W - 16)
            pltpu.sync_copy(cols_v, outT_hbm.at[:, pl.ds(base, B_PER_W)])

    return gather_kernel(user, movie, utabT, mtabT)


def _mlp_body(ueT_ref, meT_ref, w1_ref, b1_ref, w2_ref, b2_ref, w3_ref, b3_ref,
              out_ref):
    w1 = w1_ref[...]
    hT = (
        lax.dot_general(w1[:EMBED_DIM], ueT_ref[...],
                        (((0,), (0,)), ((), ())),
                        preferred_element_type=jnp.float32)
        + lax.dot_general(w1[EMBED_DIM:], meT_ref[...],
                          (((0,), (0,)), ((), ())),
                          preferred_element_type=jnp.float32)
        + b1_ref[...]
    )
    hT = jnp.maximum(hT, 0.0)
    h2T = lax.dot_general(w2_ref[...], hT, (((0,), (0,)), ((), ())),
                          preferred_element_type=jnp.float32) + b2_ref[...]
    h2T = jnp.maximum(h2T, 0.0)
    s = jnp.sum(h2T * w3_ref[...], axis=0) + b3_ref[0]
    out_ref[...] = 1.0 / (1.0 + jnp.exp(-s))


def _tc_mlp(ueT, meT, W1, b1, W2, b2, W3, b3):
    b1c = b1.reshape(HIDDEN_DIM, 1)
    b2c = b2.reshape(HIDDEN_DIM // 2, 1)
    grid = (CHUNK // BLK,)
    full = lambda shape: pl.BlockSpec(shape, lambda i: tuple(0 for _ in shape))
    return pl.pallas_call(
        _mlp_body,
        grid=grid,
        in_specs=[
            pl.BlockSpec((EMBED_DIM, BLK), lambda i: (0, i)),
            pl.BlockSpec((EMBED_DIM, BLK), lambda i: (0, i)),
            full((2 * EMBED_DIM, HIDDEN_DIM)),
            full((HIDDEN_DIM, 1)),
            full((HIDDEN_DIM, HIDDEN_DIM // 2)),
            full((HIDDEN_DIM // 2, 1)),
            full((HIDDEN_DIM // 2, 1)),
            pl.BlockSpec(memory_space=pltpu.SMEM),
        ],
        out_specs=pl.BlockSpec((BLK,), lambda i: (i,)),
        out_shape=jax.ShapeDtypeStruct((CHUNK,), jnp.float32),
    )(ueT, meT, W1, b1c, W2, b2c, W3, b3)


def kernel(user, movie, user_table, movie_table, W1, b1, W2, b2, W3, b3):
    user = user.astype(jnp.int32)
    movie = movie.astype(jnp.int32)
    utabT = user_table.T
    mtabT = movie_table.T
    embs = [
        _sc_gather(user[c * CHUNK:(c + 1) * CHUNK],
                   movie[c * CHUNK:(c + 1) * CHUNK], utabT, mtabT)
        for c in range(N_CHUNKS)
    ]
    outs = [_tc_mlp(ueT, meT, W1, b1, W2, b2, W3, b3) for ueT, meT in embs]
    return jnp.concatenate(outs)


# speedup vs baseline: 1.0371x; 1.0066x over previous
import functools

import jax
import jax.numpy as jnp
from jax import lax
from jax.experimental import pallas as pl
from jax.experimental.pallas import tpu as pltpu
from jax.experimental.pallas import tpu_sc as plsc

EMBED_DIM = 32
HIDDEN_DIM = 128
BATCH = 16384
N_CHUNKS = 1
CHUNK = BATCH // N_CHUNKS

NC = 2
NS = 16
NW = NC * NS
B_PER_W = CHUNK // NW

BLK = 4096


TILE_W = 128
HALF = 8


def _sc_gather(user, movie, utabT, mtabT):
    mesh = plsc.VectorSubcoreMesh(
        core_axis_name="c", subcore_axis_name="s", num_cores=NC, num_subcores=NS
    )
    n_pairs = B_PER_W // 16

    @functools.partial(
        pl.kernel,
        mesh=mesh,
        compiler_params=pltpu.CompilerParams(needs_layout_passes=False),
        out_type=[
            jax.ShapeDtypeStruct((EMBED_DIM, CHUNK), jnp.float32),
            jax.ShapeDtypeStruct((EMBED_DIM, CHUNK), jnp.float32),
        ],
        scratch_types=[
            pltpu.VMEM((B_PER_W,), jnp.int32),
            pltpu.VMEM((B_PER_W,), jnp.int32),
            pltpu.VMEM((16, EMBED_DIM, TILE_W), jnp.float32),
            pltpu.VMEM((EMBED_DIM, B_PER_W), jnp.float32),
            pltpu.SemaphoreType.DMA,
            pltpu.SemaphoreType.DMA,
        ],
    )
    def gather_kernel(user_hbm, movie_hbm, utabT_hbm, mtabT_hbm,
                      ueT_hbm, meT_hbm,
                      uidx_v, midx_v, bufs_v, cols_v, sem_a, sem_b):
        wid = lax.axis_index("s") * NC + lax.axis_index("c")
        base = wid * B_PER_W
        pltpu.sync_copy(user_hbm.at[pl.ds(base, B_PER_W)], uidx_v)
        pltpu.sync_copy(movie_hbm.at[pl.ds(base, B_PER_W)], midx_v)

        rows0 = lax.broadcasted_iota(jnp.int32, (16,), 0)
        rows1 = rows0 + 16
        sems = (sem_a, sem_b)

        for idx_v, tabT_hbm, outT_hbm in (
            (uidx_v, utabT_hbm, ueT_hbm),
            (midx_v, mtabT_hbm, meT_hbm),
        ):
            def fire(iv, half, tabT_hbm=tabT_hbm):
                for i in range(HALF):
                    s = iv[half * HALF + i]
                    q = pl.multiple_of((s >> 7) << 7, TILE_W)
                    pltpu.async_copy(
                        tabT_hbm.at[:, pl.ds(q, TILE_W)],
                        bufs_v.at[half * HALF + i], sems[half])

            def drain(half, tabT_hbm=tabT_hbm):
                for _ in range(HALF):
                    pltpu.make_async_copy(
                        tabT_hbm.at[:, pl.ds(0, TILE_W)],
                        bufs_v.at[0], sems[half]).wait()

            def extract(iv, half, jbase):
                for i in range(HALF):
                    s = iv[half * HALF + i]
                    lane = jnp.broadcast_to(s & 127, (16,))
                    jcol = jnp.broadcast_to(jbase + (half * HALF + i), (16,))
                    slot = bufs_v.at[half * HALF + i]
                    v0 = plsc.load_gather(slot, [rows0, lane])
                    v1 = plsc.load_gather(slot, [rows1, lane])
                    plsc.store_scatter(cols_v, [rows0, jcol], v0)
                    plsc.store_scatter(cols_v, [rows1, jcol], v1)

            iv0 = idx_v[pl.ds(0, 16)]
            fire(iv0, 0)
            fire(iv0, 1)

            def pair(p, carry, idx_v=idx_v):
                offp = pl.multiple_of(p * 16, 16)
                ivprev = idx_v[pl.ds(offp - 16, 16)]
                ivcur = idx_v[pl.ds(offp, 16)]
                drain(0)
                extract(ivprev, 0, offp - 16)
                fire(ivcur, 0)
                drain(1)
                extract(ivprev, 1, offp - 16)
                fire(ivcur, 1)
                return carry

            lax.fori_loop(1, n_pairs, pair, 0)

            ivlast = idx_v[pl.ds(B_PER_W - 16, 16)]
            drain(0)
            extract(ivlast, 0, B_PER_W - 16)
            drain(1)
            extract(ivlast, 1, B_PER_W - 16)
            pltpu.sync_copy(cols_v, outT_hbm.at[:, pl.ds(base, B_PER_W)])

    return gather_kernel(user, movie, utabT, mtabT)


def _mlp_body(ueT_ref, meT_ref, w1_ref, b1_ref, w2_ref, b2_ref, w3_ref, b3_ref,
              out_ref):
    w1 = w1_ref[...]
    hT = (
        lax.dot_general(w1[:EMBED_DIM], ueT_ref[...],
                        (((0,), (0,)), ((), ())),
                        preferred_element_type=jnp.float32)
        + lax.dot_general(w1[EMBED_DIM:], meT_ref[...],
                          (((0,), (0,)), ((), ())),
                          preferred_element_type=jnp.float32)
        + b1_ref[...]
    )
    hT = jnp.maximum(hT, 0.0)
    h2T = lax.dot_general(w2_ref[...], hT, (((0,), (0,)), ((), ())),
                          preferred_element_type=jnp.float32) + b2_ref[...]
    h2T = jnp.maximum(h2T, 0.0)
    s = jnp.sum(h2T * w3_ref[...], axis=0) + b3_ref[0]
    out_ref[...] = 1.0 / (1.0 + jnp.exp(-s))


def _tc_mlp(ueT, meT, W1, b1, W2, b2, W3, b3):
    b1c = b1.reshape(HIDDEN_DIM, 1)
    b2c = b2.reshape(HIDDEN_DIM // 2, 1)
    grid = (CHUNK // BLK,)
    full = lambda shape: pl.BlockSpec(shape, lambda i: tuple(0 for _ in shape))
    return pl.pallas_call(
        _mlp_body,
        grid=grid,
        in_specs=[
            pl.BlockSpec((EMBED_DIM, BLK), lambda i: (0, i)),
            pl.BlockSpec((EMBED_DIM, BLK), lambda i: (0, i)),
            full((2 * EMBED_DIM, HIDDEN_DIM)),
            full((HIDDEN_DIM, 1)),
            full((HIDDEN_DIM, HIDDEN_DIM // 2)),
            full((HIDDEN_DIM // 2, 1)),
            full((HIDDEN_DIM // 2, 1)),
            pl.BlockSpec(memory_space=pltpu.SMEM),
        ],
        out_specs=pl.BlockSpec((BLK,), lambda i: (i,)),
        out_shape=jax.ShapeDtypeStruct((CHUNK,), jnp.float32),
    )(ueT, meT, W1, b1c, W2, b2c, W3, b3)


def kernel(user, movie, user_table, movie_table, W1, b1, W2, b2, W3, b3):
    user = user.astype(jnp.int32)
    movie = movie.astype(jnp.int32)
    utabT = user_table.T
    mtabT = movie_table.T
    embs = [
        _sc_gather(user[c * CHUNK:(c + 1) * CHUNK],
                   movie[c * CHUNK:(c + 1) * CHUNK], utabT, mtabT)
        for c in range(N_CHUNKS)
    ]
    outs = [_tc_mlp(ueT, meT, W1, b1, W2, b2, W3, b3) for ueT, meT in embs]
    return jnp.concatenate(outs)


# speedup vs baseline: 1.0456x; 1.0081x over previous
import functools

import jax
import jax.numpy as jnp
from jax import lax
from jax.experimental import pallas as pl
from jax.experimental.pallas import tpu as pltpu
from jax.experimental.pallas import tpu_sc as plsc

EMBED_DIM = 32
HIDDEN_DIM = 128
BATCH = 16384
N_CHUNKS = 1
CHUNK = BATCH // N_CHUNKS

NC = 2
NS = 16
NW = NC * NS
B_PER_W = CHUNK // NW

BLK = 8192


TILE_W = 128
HALF = 8


def _sc_gather(user, movie, utabT, mtabT):
    mesh = plsc.VectorSubcoreMesh(
        core_axis_name="c", subcore_axis_name="s", num_cores=NC, num_subcores=NS
    )
    n_pairs = B_PER_W // 16

    @functools.partial(
        pl.kernel,
        mesh=mesh,
        compiler_params=pltpu.CompilerParams(needs_layout_passes=False),
        out_type=[
            jax.ShapeDtypeStruct((EMBED_DIM, CHUNK), jnp.float32),
            jax.ShapeDtypeStruct((EMBED_DIM, CHUNK), jnp.float32),
        ],
        scratch_types=[
            pltpu.VMEM((B_PER_W,), jnp.int32),
            pltpu.VMEM((B_PER_W,), jnp.int32),
            pltpu.VMEM((16, EMBED_DIM, TILE_W), jnp.float32),
            pltpu.VMEM((EMBED_DIM, B_PER_W), jnp.float32),
            pltpu.SemaphoreType.DMA,
            pltpu.SemaphoreType.DMA,
        ],
    )
    def gather_kernel(user_hbm, movie_hbm, utabT_hbm, mtabT_hbm,
                      ueT_hbm, meT_hbm,
                      uidx_v, midx_v, bufs_v, cols_v, sem_a, sem_b):
        wid = lax.axis_index("s") * NC + lax.axis_index("c")
        base = wid * B_PER_W
        pltpu.sync_copy(user_hbm.at[pl.ds(base, B_PER_W)], uidx_v)
        pltpu.sync_copy(movie_hbm.at[pl.ds(base, B_PER_W)], midx_v)

        rows0 = lax.broadcasted_iota(jnp.int32, (16,), 0)
        rows1 = rows0 + 16
        sems = (sem_a, sem_b)

        for idx_v, tabT_hbm, outT_hbm in (
            (uidx_v, utabT_hbm, ueT_hbm),
            (midx_v, mtabT_hbm, meT_hbm),
        ):
            def fire(iv, half, tabT_hbm=tabT_hbm):
                for i in range(HALF):
                    s = iv[half * HALF + i]
                    q = pl.multiple_of((s >> 7) << 7, TILE_W)
                    pltpu.async_copy(
                        tabT_hbm.at[:, pl.ds(q, TILE_W)],
                        bufs_v.at[half * HALF + i], sems[half])

            def drain(half, tabT_hbm=tabT_hbm):
                for _ in range(HALF):
                    pltpu.make_async_copy(
                        tabT_hbm.at[:, pl.ds(0, TILE_W)],
                        bufs_v.at[0], sems[half]).wait()

            def extract(iv, half, jbase):
                for i in range(HALF):
                    s = iv[half * HALF + i]
                    lane = jnp.broadcast_to(s & 127, (16,))
                    jcol = jnp.broadcast_to(jbase + (half * HALF + i), (16,))
                    slot = bufs_v.at[half * HALF + i]
                    v0 = plsc.load_gather(slot, [rows0, lane])
                    v1 = plsc.load_gather(slot, [rows1, lane])
                    plsc.store_scatter(cols_v, [rows0, jcol], v0)
                    plsc.store_scatter(cols_v, [rows1, jcol], v1)

            iv0 = idx_v[pl.ds(0, 16)]
            fire(iv0, 0)
            fire(iv0, 1)

            def pair(p, carry, idx_v=idx_v):
                offp = pl.multiple_of(p * 16, 16)
                ivprev = idx_v[pl.ds(offp - 16, 16)]
                ivcur = idx_v[pl.ds(offp, 16)]
                drain(0)
                extract(ivprev, 0, offp - 16)
                fire(ivcur, 0)
                drain(1)
                extract(ivprev, 1, offp - 16)
                fire(ivcur, 1)
                return carry

            lax.fori_loop(1, n_pairs, pair, 0)

            ivlast = idx_v[pl.ds(B_PER_W - 16, 16)]
            drain(0)
            extract(ivlast, 0, B_PER_W - 16)
            drain(1)
            extract(ivlast, 1, B_PER_W - 16)
            pltpu.sync_copy(cols_v, outT_hbm.at[:, pl.ds(base, B_PER_W)])

    return gather_kernel(user, movie, utabT, mtabT)


def _mlp_body(ueT_ref, meT_ref, w1_ref, b1_ref, w2_ref, b2_ref, w3_ref, b3_ref,
              out_ref):
    w1 = w1_ref[...]
    hT = (
        lax.dot_general(w1[:EMBED_DIM], ueT_ref[...],
                        (((0,), (0,)), ((), ())),
                        preferred_element_type=jnp.float32)
        + lax.dot_general(w1[EMBED_DIM:], meT_ref[...],
                          (((0,), (0,)), ((), ())),
                          preferred_element_type=jnp.float32)
        + b1_ref[...]
    )
    hT = jnp.maximum(hT, 0.0)
    h2T = lax.dot_general(w2_ref[...], hT, (((0,), (0,)), ((), ())),
                          preferred_element_type=jnp.float32) + b2_ref[...]
    h2T = jnp.maximum(h2T, 0.0)
    s = jnp.sum(h2T * w3_ref[...], axis=0) + b3_ref[0]
    out_ref[...] = 1.0 / (1.0 + jnp.exp(-s))


def _tc_mlp(ueT, meT, W1, b1, W2, b2, W3, b3):
    b1c = b1.reshape(HIDDEN_DIM, 1)
    b2c = b2.reshape(HIDDEN_DIM // 2, 1)
    grid = (CHUNK // BLK,)
    full = lambda shape: pl.BlockSpec(shape, lambda i: tuple(0 for _ in shape))
    return pl.pallas_call(
        _mlp_body,
        grid=grid,
        in_specs=[
            pl.BlockSpec((EMBED_DIM, BLK), lambda i: (0, i)),
            pl.BlockSpec((EMBED_DIM, BLK), lambda i: (0, i)),
            full((2 * EMBED_DIM, HIDDEN_DIM)),
            full((HIDDEN_DIM, 1)),
            full((HIDDEN_DIM, HIDDEN_DIM // 2)),
            full((HIDDEN_DIM // 2, 1)),
            full((HIDDEN_DIM // 2, 1)),
            pl.BlockSpec(memory_space=pltpu.SMEM),
        ],
        out_specs=pl.BlockSpec((BLK,), lambda i: (i,)),
        out_shape=jax.ShapeDtypeStruct((CHUNK,), jnp.float32),
    )(ueT, meT, W1, b1c, W2, b2c, W3, b3)


def kernel(user, movie, user_table, movie_table, W1, b1, W2, b2, W3, b3):
    user = user.astype(jnp.int32)
    movie = movie.astype(jnp.int32)
    utabT = user_table.T
    mtabT = movie_table.T
    embs = [
        _sc_gather(user[c * CHUNK:(c + 1) * CHUNK],
                   movie[c * CHUNK:(c + 1) * CHUNK], utabT, mtabT)
        for c in range(N_CHUNKS)
    ]
    outs = [_tc_mlp(ueT, meT, W1, b1, W2, b2, W3, b3) for ueT, meT in embs]
    return jnp.concatenate(outs)
